# baseline (speedup 1.0000x reference)
import functools

import jax
import jax.numpy as jnp
from jax import lax
from jax.experimental import pallas as pl
from jax.experimental.pallas import tpu as pltpu
from jax.experimental.pallas import tpu_sc as plsc

B, S, D = 4, 2048, 1024
_NC = 2
_NW = 32
_S_SC = 512
_SPW = _S_SC // _NW
_TROWS = 16
_NT = _SPW // _TROWS


@functools.partial(
    pl.kernel,
    mesh=plsc.VectorSubcoreMesh(core_axis_name="c", subcore_axis_name="s"),
    out_type=jax.ShapeDtypeStruct((B, S, D), jnp.float32),
    scratch_types=[
        pltpu.VMEM((_SPW, D), jnp.float32),
        pltpu.VMEM((_TROWS, D), jnp.float32),
        pltpu.VMEM((_TROWS, D), jnp.float32),
        pltpu.VMEM((_TROWS, D), jnp.float32),
        pltpu.SemaphoreType.DMA,
        pltpu.SemaphoreType.DMA,
        pltpu.SemaphoreType.DMA,
        pltpu.SemaphoreType.DMA,
        pltpu.SemaphoreType.DMA,
        pltpu.SemaphoreType.DMA,
    ],
)
def _sc_add(x_hbm, pe_hbm, out_hbm, pe_v, xa, xb, xc, sia, sib, sic, soa, sob, soc):
    wid = lax.axis_index("s") * _NC + lax.axis_index("c")
    base = wid * _SPW
    tiles = [(b, t) for b in range(B) for t in range(_NT)]
    bufs = [(xa, sia, soa), (xb, sib, sob), (xc, sic, soc)]
    n = len(tiles)
    in_dma = [None, None, None]
    out_dma = [None, None, None]
    b0, t0 = tiles[0]
    in_dma[0] = pltpu.async_copy(
        x_hbm.at[b0, pl.ds(base + t0 * _TROWS, _TROWS)], xa, sia
    )
    pltpu.sync_copy(pe_hbm.at[pl.ds(base, _SPW)], pe_v)
    for k, (b, t) in enumerate(tiles):
        cur = k % 3
        buf, _, sout = bufs[cur]
        in_dma[cur].wait()
        if k + 1 < n:
            nb, nt = tiles[k + 1]
            nxt = (k + 1) % 3
            nbuf, nsin, _ = bufs[nxt]
            if out_dma[nxt] is not None:
                out_dma[nxt].wait()
            in_dma[nxt] = pltpu.async_copy(
                x_hbm.at[nb, pl.ds(base + nt * _TROWS, _TROWS)], nbuf, nsin
            )

        @plsc.parallel_loop(0, _TROWS * D, step=16, unroll=16)
        def add_body(i, buf=buf, t=t):
            r = i >> 10
            c = pl.multiple_of(i & (D - 1), 16)
            buf[r, pl.ds(c, 16)] = (
                buf[r, pl.ds(c, 16)] + pe_v[t * _TROWS + r, pl.ds(c, 16)]
            )

        out_dma[cur] = pltpu.async_copy(
            buf, out_hbm.at[b, pl.ds(base + t * _TROWS, _TROWS)], sout
        )
    for d in out_dma:
        if d is not None:
            d.wait()


_BS = 256
_SC_BLOCKS = _S_SC // _BS


def _tc_add_body(x_ref, pe_ref, alias_ref, o_ref):
    del alias_ref
    o_ref[...] = x_ref[...] + pe_ref[...]


def _tc_add_rest(x, pe, sc_out):
    return pl.pallas_call(
        _tc_add_body,
        grid=(S // _BS - _SC_BLOCKS, B),
        in_specs=[
            pl.BlockSpec((1, _BS, D), lambda i, j: (j, i + _SC_BLOCKS, 0)),
            pl.BlockSpec((_BS, D), lambda i, j: (i + _SC_BLOCKS, 0)),
            pl.BlockSpec(memory_space=pl.ANY),
        ],
        out_specs=pl.BlockSpec((1, _BS, D), lambda i, j: (j, i + _SC_BLOCKS, 0)),
        out_shape=jax.ShapeDtypeStruct((B, S, D), jnp.float32),
        input_output_aliases={2: 0},
    )(x, pe, sc_out)


def kernel(x, pe):
    pe = pe[:S]
    sc_out = _sc_add(x, pe)
    return _tc_add_rest(x, pe, sc_out)


# speedup vs baseline: 1.0601x; 1.0601x over previous
import functools

import jax
import jax.numpy as jnp
from jax import lax
from jax.experimental import pallas as pl
from jax.experimental.pallas import tpu as pltpu
from jax.experimental.pallas import tpu_sc as plsc

B, S, D = 4, 2048, 1024
_NC = 2
_NW = 32
_S_SC = 2048
_SPW = _S_SC // _NW
_TROWS = 16
_NT = _SPW // _TROWS


@functools.partial(
    pl.kernel,
    mesh=plsc.VectorSubcoreMesh(core_axis_name="c", subcore_axis_name="s"),
    out_type=jax.ShapeDtypeStruct((B, S, D), jnp.float32),
    scratch_types=[
        pltpu.VMEM((_SPW, D), jnp.float32),
        pltpu.VMEM((_TROWS, D), jnp.float32),
        pltpu.VMEM((_TROWS, D), jnp.float32),
        pltpu.VMEM((_TROWS, D), jnp.float32),
        pltpu.SemaphoreType.DMA,
        pltpu.SemaphoreType.DMA,
        pltpu.SemaphoreType.DMA,
        pltpu.SemaphoreType.DMA,
        pltpu.SemaphoreType.DMA,
        pltpu.SemaphoreType.DMA,
    ],
)
def _sc_add(x_hbm, pe_hbm, out_hbm, pe_v, xa, xb, xc, sia, sib, sic, soa, sob, soc):
    wid = lax.axis_index("s") * _NC + lax.axis_index("c")
    base = wid * _SPW
    tiles = [(b, t) for b in range(B) for t in range(_NT)]
    bufs = [(xa, sia, soa), (xb, sib, sob), (xc, sic, soc)]
    n = len(tiles)
    in_dma = [None, None, None]
    out_dma = [None, None, None]
    b0, t0 = tiles[0]
    in_dma[0] = pltpu.async_copy(
        x_hbm.at[b0, pl.ds(base + t0 * _TROWS, _TROWS)], xa, sia
    )
    pltpu.sync_copy(pe_hbm.at[pl.ds(base, _SPW)], pe_v)
    for k, (b, t) in enumerate(tiles):
        cur = k % 3
        buf, _, sout = bufs[cur]
        in_dma[cur].wait()
        if k + 1 < n:
            nb, nt = tiles[k + 1]
            nxt = (k + 1) % 3
            nbuf, nsin, _ = bufs[nxt]
            if out_dma[nxt] is not None:
                out_dma[nxt].wait()
            in_dma[nxt] = pltpu.async_copy(
                x_hbm.at[nb, pl.ds(base + nt * _TROWS, _TROWS)], nbuf, nsin
            )

        @plsc.parallel_loop(0, _TROWS * D, step=16, unroll=16)
        def add_body(i, buf=buf, t=t):
            r = i >> 10
            c = pl.multiple_of(i & (D - 1), 16)
            buf[r, pl.ds(c, 16)] = (
                buf[r, pl.ds(c, 16)] + pe_v[t * _TROWS + r, pl.ds(c, 16)]
            )

        out_dma[cur] = pltpu.async_copy(
            buf, out_hbm.at[b, pl.ds(base + t * _TROWS, _TROWS)], sout
        )
    for d in out_dma:
        if d is not None:
            d.wait()


_BS = 256
_SC_BLOCKS = _S_SC // _BS


def _tc_add_body(x_ref, pe_ref, alias_ref, o_ref):
    del alias_ref
    o_ref[...] = x_ref[...] + pe_ref[...]


def _tc_add_rest(x, pe, sc_out):
    return pl.pallas_call(
        _tc_add_body,
        grid=(S // _BS - _SC_BLOCKS, B),
        in_specs=[
            pl.BlockSpec((1, _BS, D), lambda i, j: (j, i + _SC_BLOCKS, 0)),
            pl.BlockSpec((_BS, D), lambda i, j: (i + _SC_BLOCKS, 0)),
            pl.BlockSpec(memory_space=pl.ANY),
        ],
        out_specs=pl.BlockSpec((1, _BS, D), lambda i, j: (j, i + _SC_BLOCKS, 0)),
        out_shape=jax.ShapeDtypeStruct((B, S, D), jnp.float32),
        input_output_aliases={2: 0},
    )(x, pe, sc_out)


def kernel(x, pe):
    pe = pe[:S]
    sc_out = _sc_add(x, pe)
    if _S_SC == S:
        return sc_out
    return _tc_add_rest(x, pe, sc_out)


# speedup vs baseline: 1.1503x; 1.0851x over previous
import functools

import jax
import jax.numpy as jnp
from jax import lax
from jax.experimental import pallas as pl
from jax.experimental.pallas import tpu as pltpu
from jax.experimental.pallas import tpu_sc as plsc

B, S, D = 4, 2048, 1024
_NC = 2
_NW = 32
_SPW = S // _NW
_GROWS = 8
_PEROWS = 32
_GPC = _PEROWS // _GROWS
_NG = _SPW // _GROWS

_XBUF = pltpu.VMEM((_GROWS, D), jnp.float32)
_DSEM = pltpu.SemaphoreType.DMA


@functools.partial(
    pl.kernel,
    mesh=plsc.VectorSubcoreMesh(core_axis_name="c", subcore_axis_name="s"),
    out_type=jax.ShapeDtypeStruct((B, S, D), jnp.float32),
    scratch_types=(
        [pltpu.VMEM((_PEROWS, D), jnp.float32)]
        + [_XBUF] * (2 * B)
        + [_DSEM] * (2 * B)
        + [_DSEM] * (2 * B)
    ),
)
def _sc_add(x_hbm, pe_hbm, out_hbm, pe_v, *bufs_and_sems):
    xbufs = bufs_and_sems[: 2 * B]
    sins = bufs_and_sems[2 * B : 4 * B]
    souts = bufs_and_sems[4 * B : 6 * B]
    sets = [
        (xbufs[:B], sins[:B], souts[:B]),
        (xbufs[B:], sins[B:], souts[B:]),
    ]
    wid = lax.axis_index("s") * _NC + lax.axis_index("c")
    base = wid * _SPW
    in_dma = [None, None]
    out_dma = [None, None]

    def issue_in(k, si):
        xb, sin, _ = sets[si]
        row = base + k * _GROWS
        return [
            pltpu.async_copy(x_hbm.at[b, pl.ds(row, _GROWS)], xb[b], sin[b])
            for b in range(B)
        ]

    in_dma[0] = issue_in(0, 0)
    pltpu.sync_copy(pe_hbm.at[pl.ds(base, _PEROWS)], pe_v)
    for k in range(_NG):
        si = k % 2
        xb, _, sout = sets[si]
        if k > 0 and k % _GPC == 0:
            pltpu.sync_copy(
                pe_hbm.at[pl.ds(base + (k // _GPC) * _PEROWS, _PEROWS)], pe_v
            )
        for d in in_dma[si]:
            d.wait()
        if k + 1 < _NG:
            nxt = (k + 1) % 2
            if out_dma[nxt] is not None:
                for d in out_dma[nxt]:
                    d.wait()
            in_dma[nxt] = issue_in(k + 1, nxt)

        prow = (k % _GPC) * _GROWS

        @plsc.parallel_loop(0, _GROWS * D, step=16, unroll=4)
        def add_body(i, xb=xb, prow=prow):
            r = i >> 10
            c = pl.multiple_of(i & (D - 1), 16)
            pv = pe_v[prow + r, pl.ds(c, 16)]
            for b in range(B):
                xb[b][r, pl.ds(c, 16)] = xb[b][r, pl.ds(c, 16)] + pv

        row = base + k * _GROWS
        out_dma[si] = [
            pltpu.async_copy(xb[b], out_hbm.at[b, pl.ds(row, _GROWS)], sout[b])
            for b in range(B)
        ]
    for dl in out_dma:
        if dl is not None:
            for d in dl:
                d.wait()


def kernel(x, pe):
    return _sc_add(x, pe[:S])
